# baseline (speedup 1.0000x reference)
import functools

import jax
import jax.numpy as jnp
from jax import lax
from jax.experimental import pallas as pl
from jax.experimental.pallas import tpu as pltpu
from jax.experimental.pallas import tpu_sc as plsc

_F32 = jnp.float32
_CH = 128
_NB = 4
_NB5 = 2


def _sc_mesh():
    return plsc.VectorSubcoreMesh(core_axis_name="c", subcore_axis_name="s")


def _zero_fill_1d(ref, nvec):
    def body(i, carry):
        ref[pl.ds(i * 16, 16)] = jnp.zeros((16,), _F32)
        return carry
    lax.fori_loop(0, nvec, body, 0)


def _newton_rsqrt(x):
    bits = lax.bitcast_convert_type(x, jnp.int32)
    seed = lax.bitcast_convert_type(
        0x5F3759DF - lax.shift_right_logical(bits, 1), _F32)
    y = seed * (1.5 - 0.5 * x * seed * seed)
    y = y * (1.5 - 0.5 * x * y * y)
    return y


def _sc_norm(NP, NCH16, NCH32):
    rpt = NP // 16

    @functools.partial(
        pl.kernel,
        mesh=_sc_mesh(),
        out_type=(jax.ShapeDtypeStruct((2 * NP,), _F32),
                  jax.ShapeDtypeStruct((2 * NP,), _F32)),
        scratch_types=[
            pltpu.VMEM((NCH16, _CH), jnp.int32),
            pltpu.VMEM((NCH32, _CH), jnp.int32),
            pltpu.VMEM((NCH32, _CH), jnp.int32),
            pltpu.VMEM((_CH,), _F32),
            pltpu.VMEM((rpt,), _F32),
            pltpu.VMEM((rpt,), _F32),
            [pltpu.VMEM((_CH,), _F32)] * _NB,
            pltpu.VMEM_SHARED((NP,), _F32),
            [pltpu.SemaphoreType.DMA] * _NB,
        ],
    )
    def k(srcc16, srcc32o, dstc32, dinv_out, sp_out,
          idxh, isrc, idst, ones_v, zeros_v, degv, vrows, acc, sems):
        cid = lax.axis_index("c")
        sid = lax.axis_index("s")
        wid = sid * 2 + cid
        hid = cid * 16 + sid
        for t in range(_CH // 16):
            ones_v[pl.ds(t * 16, 16)] = jnp.ones((16,), _F32)
        _zero_fill_1d(zeros_v, rpt // 16)
        pltpu.sync_copy(zeros_v, acc.at[pl.ds(sid * rpt, rpt)])
        pltpu.sync_copy(srcc16.at[hid], idxh)
        pltpu.sync_copy(srcc32o.at[wid], isrc)
        pltpu.sync_copy(dstc32.at[wid], idst)
        plsc.subcore_barrier()

        def hbody(j, carry):
            pltpu.sync_copy(ones_v, acc.at[idxh.at[j]], add=True)
            return carry
        lax.fori_loop(0, NCH16, hbody, 0)
        plsc.subcore_barrier()

        pltpu.sync_copy(acc.at[pl.ds(sid * rpt, rpt)], degv)

        def rbody(i, carry):
            x = degv[pl.ds(i * 16, 16)] + 1.0
            degv[pl.ds(i * 16, 16)] = _newton_rsqrt(x)
            return carry
        lax.fori_loop(0, rpt // 16, rbody, 0)
        pltpu.sync_copy(
            degv, dinv_out.at[pl.ds(cid * NP + sid * rpt, rpt)])
        pltpu.sync_copy(zeros_v, acc.at[pl.ds(sid * rpt, rpt)])
        plsc.subcore_barrier()

        for b in range(_NB):
            pltpu.async_copy(dinv_out.at[isrc.at[b]], vrows[b], sems[b])

        def sbody(q, carry):
            for b in range(_NB):
                e = q * _NB + b
                pltpu.make_async_copy(
                    dinv_out.at[pl.ds(0, _CH)], vrows[b], sems[b]).wait()
                pltpu.sync_copy(vrows[b], acc.at[idst.at[e]], add=True)
                nxt = jnp.minimum(e + _NB, NCH32 - 1)
                pltpu.async_copy(
                    dinv_out.at[isrc.at[nxt]], vrows[b], sems[b])
            return carry
        lax.fori_loop(0, NCH32 // _NB, sbody, 0)
        for b in range(_NB):
            pltpu.make_async_copy(
                dinv_out.at[pl.ds(0, _CH)], vrows[b], sems[b]).wait()
        plsc.subcore_barrier()
        pltpu.sync_copy(acc.at[pl.ds(sid * rpt, rpt)],
                        sp_out.at[pl.ds(cid * NP + sid * rpt, rpt)])

    return k


def _sc_aggregate(NP, NCH, D):
    rpt = NP // 16
    nzc = rpt // _CH

    @functools.partial(
        pl.kernel,
        mesh=_sc_mesh(),
        out_type=jax.ShapeDtypeStruct((2 * NP, D), _F32),
        scratch_types=[
            pltpu.VMEM((NCH, _CH), jnp.int32),
            pltpu.VMEM((NCH, _CH), jnp.int32),
            pltpu.VMEM((_CH, D), _F32),
            pltpu.VMEM_SHARED((NP, D), _F32),
            pltpu.SemaphoreType.DMA,
        ],
    )
    def k(srcc, dstc, xws, out, isrc, idst, rows, acc, gsem):
        cid = lax.axis_index("c")
        sid = lax.axis_index("s")
        wid = sid * 2 + cid
        pltpu.sync_copy(srcc.at[wid], isrc)
        pltpu.sync_copy(dstc.at[wid], idst)

        def zrow(i, carry):
            for t in range(D // 16):
                rows[i, pl.ds(t * 16, 16)] = jnp.zeros((16,), _F32)
            return carry
        lax.fori_loop(0, _CH, zrow, 0)
        for t in range(nzc):
            pltpu.sync_copy(
                rows, acc.at[pl.ds(sid * rpt + t * _CH, _CH), :])
        plsc.subcore_barrier()

        def body(j, carry):
            pltpu.async_copy(xws.at[isrc.at[j]], rows, gsem).wait()
            pltpu.sync_copy(rows, acc.at[idst.at[j]], add=True)
            return carry
        lax.fori_loop(0, NCH, body, 0)
        plsc.subcore_barrier()
        for t in range(nzc):
            pltpu.sync_copy(
                acc.at[pl.ds(sid * rpt + t * _CH, _CH), :],
                out.at[pl.ds(cid * NP + sid * rpt + t * _CH, _CH), :])

    return k


def _tc_chain(NP, D):
    R = NP // D

    def body(dinv_ref, sp_ref, c_ref, q_ref):
        dinv = dinv_ref[...]
        s = sp_ref[0] + sp_ref[1]
        deg2 = dinv * s + dinv * dinv + 2.0
        dinv2 = lax.rsqrt(deg2)
        c = dinv * dinv2
        c_ref[...] = c
        q_ref[...] = (c * c + 2.0 * dinv2 * dinv2) / c

    return pl.pallas_call(
        body,
        out_shape=(jax.ShapeDtypeStruct((R, D), _F32),
                   jax.ShapeDtypeStruct((R, D), _F32)),
    )


def _tc_scale_mm(NP, n, D):
    def body(x_ref, w_ref, c_ref, xws_ref):
        xw = lax.dot_general(
            x_ref[...], w_ref[...], (((1,), (1,)), ((), ())),
            precision=lax.Precision.HIGHEST,
            preferred_element_type=_F32)
        xws_ref[pl.ds(0, n), :] = xw * c_ref[pl.ds(0, n), :]
        xws_ref[pl.ds(n, NP - n), :] = jnp.zeros((NP - n, D), _F32)

    return pl.pallas_call(
        body,
        out_shape=jax.ShapeDtypeStruct((NP, D), _F32),
    )


def _tc_epilogue(NP, n, D):
    def body(agg_ref, xws_ref, x_ref, c_ref, q_ref,
             b_ref, g_ref, be_ref, y_ref):
        agg = agg_ref[pl.ds(0, n), :] + agg_ref[pl.ds(NP, n), :]
        out = (c_ref[pl.ds(0, n), :] * agg
               + q_ref[pl.ds(0, n), :] * xws_ref[pl.ds(0, n), :]
               + b_ref[...])
        mean = jnp.mean(out, axis=0, keepdims=True)
        var = jnp.mean((out - mean) ** 2, axis=0, keepdims=True)
        h = (out - mean) * lax.rsqrt(var + 1e-5) * g_ref[...] + be_ref[...]
        h = jnp.where(h > 0, h, 0.1 * h)
        y = h + x_ref[...]
        y_ref[...] = jnp.where(y > 0, y, 0.1 * y)

    return pl.pallas_call(
        body,
        out_shape=jax.ShapeDtypeStruct((n, D), _F32),
    )


def kernel(x, edge_index, W, b, gamma, beta):
    n, d = x.shape
    dh = d // 2
    e = edge_index.shape[1]
    grain32 = 32 * _CH * _NB
    EP = -(-e // grain32) * grain32
    NCH32 = EP // (32 * _CH)
    NCH16 = EP // (16 * _CH)
    NP = -(-(n + 16) // 256) * 256

    pad = EP - e
    padv = n + (jnp.arange(pad, dtype=jnp.int32) % 16)
    src = jnp.concatenate([edge_index[0], padv])
    dst = jnp.concatenate([edge_index[1], padv])
    srcc32 = src.reshape(32, NCH32, _CH)
    dstc32 = dst.reshape(32, NCH32, _CH)

    NCH16 = EP // (16 * _CH)
    src16 = src.reshape(1, 16, NCH16, _CH)
    srcc16 = jnp.concatenate([src16, src16], axis=0).reshape(
        32, NCH16, _CH)
    coff = (jnp.arange(32, dtype=jnp.int32)[:, None, None] % 2) * NP
    srcc32o = srcc32 + coff
    dinv2x, sp = _sc_norm(NP, NCH16, NCH32)(srcc16, srcc32o, dstc32)
    R = NP // d
    c2d, q2d = _tc_chain(NP, d)(
        dinv2x[:NP].reshape(R, d), sp.reshape(2, R, d))
    c_col = c2d.reshape(NP, 1)
    q_col = q2d.reshape(NP, 1)
    xws = _tc_scale_mm(NP, n, d)(x, W, c_col)
    aggp = _sc_aggregate(NP, NCH32, d)(srcc32, dstc32, xws)
    y = _tc_epilogue(NP, n, d)(
        aggp, xws, x, c_col, q_col,
        b.reshape(1, d), gamma.reshape(1, d), beta.reshape(1, d))
    return y


# speedup vs baseline: 1.2535x; 1.2535x over previous
import functools

import jax
import jax.numpy as jnp
from jax import lax
from jax.experimental import pallas as pl
from jax.experimental.pallas import tpu as pltpu
from jax.experimental.pallas import tpu_sc as plsc

_F32 = jnp.float32
_CH = 128
_NB = 4
_NB5 = 2


def _sc_mesh():
    return plsc.VectorSubcoreMesh(core_axis_name="c", subcore_axis_name="s")


def _zero_fill_1d(ref, nvec):
    def body(i, carry):
        ref[pl.ds(i * 16, 16)] = jnp.zeros((16,), _F32)
        return carry
    lax.fori_loop(0, nvec, body, 0)


def _newton_rsqrt(x):
    bits = lax.bitcast_convert_type(x, jnp.int32)
    seed = lax.bitcast_convert_type(
        0x5F3759DF - lax.shift_right_logical(bits, 1), _F32)
    y = seed * (1.5 - 0.5 * x * seed * seed)
    y = y * (1.5 - 0.5 * x * y * y)
    return y


def _sc_norm(NP, NCH16, NCH32):
    rpt = NP // 16

    @functools.partial(
        pl.kernel,
        mesh=_sc_mesh(),
        out_type=(jax.ShapeDtypeStruct((2 * NP,), _F32),
                  jax.ShapeDtypeStruct((2 * NP,), _F32)),
        scratch_types=[
            pltpu.VMEM((NCH16, _CH), jnp.int32),
            pltpu.VMEM((NCH32, _CH), jnp.int32),
            pltpu.VMEM((NCH32, _CH), jnp.int32),
            pltpu.VMEM((_CH,), _F32),
            pltpu.VMEM((rpt,), _F32),
            pltpu.VMEM((rpt,), _F32),
            [pltpu.VMEM((_CH,), _F32)] * _NB,
            pltpu.VMEM_SHARED((NP,), _F32),
            [pltpu.SemaphoreType.DMA] * _NB,
        ],
    )
    def k(srcc16, srcc32o, dstc32, dinv_out, sp_out,
          idxh, isrc, idst, ones_v, zeros_v, degv, vrows, acc, sems):
        cid = lax.axis_index("c")
        sid = lax.axis_index("s")
        wid = sid * 2 + cid
        hid = cid * 16 + sid
        for t in range(_CH // 16):
            ones_v[pl.ds(t * 16, 16)] = jnp.ones((16,), _F32)
        _zero_fill_1d(zeros_v, rpt // 16)
        pltpu.sync_copy(zeros_v, acc.at[pl.ds(sid * rpt, rpt)])
        pltpu.sync_copy(srcc16.at[hid], idxh)
        pltpu.sync_copy(srcc32o.at[wid], isrc)
        pltpu.sync_copy(dstc32.at[wid], idst)
        plsc.subcore_barrier()

        def hbody(j, carry):
            pltpu.sync_copy(ones_v, acc.at[idxh.at[j]], add=True)
            return carry
        lax.fori_loop(0, NCH16, hbody, 0)
        plsc.subcore_barrier()

        pltpu.sync_copy(acc.at[pl.ds(sid * rpt, rpt)], degv)

        def rbody(i, carry):
            x = degv[pl.ds(i * 16, 16)] + 1.0
            degv[pl.ds(i * 16, 16)] = _newton_rsqrt(x)
            return carry
        lax.fori_loop(0, rpt // 16, rbody, 0)
        pltpu.sync_copy(
            degv, dinv_out.at[pl.ds(cid * NP + sid * rpt, rpt)])
        pltpu.sync_copy(zeros_v, acc.at[pl.ds(sid * rpt, rpt)])
        plsc.subcore_barrier()

        for b in range(_NB):
            pltpu.async_copy(dinv_out.at[isrc.at[b]], vrows[b], sems[b])

        def sbody(q, carry):
            for b in range(_NB):
                e = q * _NB + b
                pltpu.make_async_copy(
                    dinv_out.at[pl.ds(0, _CH)], vrows[b], sems[b]).wait()
                pltpu.sync_copy(vrows[b], acc.at[idst.at[e]], add=True)
                nxt = jnp.minimum(e + _NB, NCH32 - 1)
                pltpu.async_copy(
                    dinv_out.at[isrc.at[nxt]], vrows[b], sems[b])
            return carry
        lax.fori_loop(0, NCH32 // _NB, sbody, 0)
        for b in range(_NB):
            pltpu.make_async_copy(
                dinv_out.at[pl.ds(0, _CH)], vrows[b], sems[b]).wait()
        plsc.subcore_barrier()
        pltpu.sync_copy(acc.at[pl.ds(sid * rpt, rpt)],
                        sp_out.at[pl.ds(cid * NP + sid * rpt, rpt)])

    return k


def _sc_aggregate(NP, NCH, D):
    rpt = NP // 16
    nzc = rpt // _CH
    HCH = NCH // 2

    @functools.partial(
        pl.kernel,
        mesh=_sc_mesh(),
        out_type=jax.ShapeDtypeStruct((2 * NP, D), _F32),
        compiler_params=pltpu.CompilerParams(use_tc_tiling_on_sc=False),
        scratch_types=[
            pltpu.VMEM((HCH, _CH), jnp.int32),
            pltpu.VMEM((HCH, _CH), jnp.int32),
            [pltpu.VMEM((_CH, D), _F32)] * 2,
            pltpu.VMEM_SHARED((NP, D), _F32),
            [pltpu.SemaphoreType.DMA] * 2,
        ],
    )
    def k(srcc, dstc, xws, out, isrc, idst, rows2, acc, gsems):
        cid = lax.axis_index("c")
        sid = lax.axis_index("s")
        wid = sid * 2 + cid

        def zrow(i, carry):
            for t in range(D // 16):
                rows2[0][i, pl.ds(t * 16, 16)] = jnp.zeros((16,), _F32)
            return carry
        lax.fori_loop(0, _CH, zrow, 0)
        for t in range(nzc):
            pltpu.sync_copy(
                rows2[0], acc.at[pl.ds(sid * rpt + t * _CH, _CH), :])
        plsc.subcore_barrier()

        for p in range(2):
            pltpu.sync_copy(srcc.at[wid, p], isrc)
            pltpu.sync_copy(dstc.at[wid, p], idst)
            for b in range(2):
                pltpu.async_copy(xws.at[isrc.at[b]], rows2[b], gsems[b])

            def body(q, carry):
                for b in range(2):
                    e = q * 2 + b
                    pltpu.make_async_copy(
                        xws.at[pl.ds(0, _CH), :], rows2[b],
                        gsems[b]).wait()
                    pltpu.sync_copy(rows2[b], acc.at[idst.at[e]], add=True)
                    nxt = jnp.minimum(e + 2, HCH - 1)
                    pltpu.async_copy(
                        xws.at[isrc.at[nxt]], rows2[b], gsems[b])
                return carry
            lax.fori_loop(0, HCH // 2, body, 0)
            for b in range(2):
                pltpu.make_async_copy(
                    xws.at[pl.ds(0, _CH), :], rows2[b], gsems[b]).wait()
        plsc.subcore_barrier()
        for t in range(nzc):
            pltpu.sync_copy(
                acc.at[pl.ds(sid * rpt + t * _CH, _CH), :],
                out.at[pl.ds(cid * NP + sid * rpt + t * _CH, _CH), :])

    return k


def _tc_chain(NP, D):
    R = NP // D

    def body(dinv_ref, sp_ref, c_ref, q_ref):
        dinv = dinv_ref[...]
        s = sp_ref[0] + sp_ref[1]
        deg2 = dinv * s + dinv * dinv + 2.0
        dinv2 = lax.rsqrt(deg2)
        c = dinv * dinv2
        c_ref[...] = c
        q_ref[...] = (c * c + 2.0 * dinv2 * dinv2) / c

    return pl.pallas_call(
        body,
        out_shape=(jax.ShapeDtypeStruct((R, D), _F32),
                   jax.ShapeDtypeStruct((R, D), _F32)),
    )


def _tc_scale_mm(NP, n, D):
    def body(x_ref, w_ref, c_ref, xws_ref):
        xw = lax.dot_general(
            x_ref[...], w_ref[...], (((1,), (1,)), ((), ())),
            precision=lax.Precision.HIGHEST,
            preferred_element_type=_F32)
        xws_ref[pl.ds(0, n), :] = xw * c_ref[pl.ds(0, n), :]
        xws_ref[pl.ds(n, NP - n), :] = jnp.zeros((NP - n, D), _F32)

    return pl.pallas_call(
        body,
        out_shape=jax.ShapeDtypeStruct((NP, D), _F32),
    )


def _tc_epilogue(NP, n, D):
    def body(agg_ref, xws_ref, x_ref, c_ref, q_ref,
             b_ref, g_ref, be_ref, y_ref):
        agg = agg_ref[pl.ds(0, n), :] + agg_ref[pl.ds(NP, n), :]
        out = (c_ref[pl.ds(0, n), :] * agg
               + q_ref[pl.ds(0, n), :] * xws_ref[pl.ds(0, n), :]
               + b_ref[...])
        mean = jnp.mean(out, axis=0, keepdims=True)
        var = jnp.mean((out - mean) ** 2, axis=0, keepdims=True)
        h = (out - mean) * lax.rsqrt(var + 1e-5) * g_ref[...] + be_ref[...]
        h = jnp.where(h > 0, h, 0.1 * h)
        y = h + x_ref[...]
        y_ref[...] = jnp.where(y > 0, y, 0.1 * y)

    return pl.pallas_call(
        body,
        out_shape=jax.ShapeDtypeStruct((n, D), _F32),
    )


def kernel(x, edge_index, W, b, gamma, beta):
    n, d = x.shape
    dh = d // 2
    e = edge_index.shape[1]
    grain32 = 32 * _CH * _NB
    EP = -(-e // grain32) * grain32
    NCH32 = EP // (32 * _CH)
    NCH16 = EP // (16 * _CH)
    NP = -(-(n + 16) // 256) * 256

    pad = EP - e
    padv = n + (jnp.arange(pad, dtype=jnp.int32) % 16)
    src = jnp.concatenate([edge_index[0], padv])
    dst = jnp.concatenate([edge_index[1], padv])
    srcc32 = src.reshape(32, NCH32, _CH)
    dstc32 = dst.reshape(32, NCH32, _CH)

    NCH16 = EP // (16 * _CH)
    src16 = src.reshape(1, 16, NCH16, _CH)
    srcc16 = jnp.concatenate([src16, src16], axis=0).reshape(
        32, NCH16, _CH)
    coff = (jnp.arange(32, dtype=jnp.int32)[:, None, None] % 2) * NP
    srcc32o = srcc32 + coff
    dinv2x, sp = _sc_norm(NP, NCH16, NCH32)(srcc16, srcc32o, dstc32)
    R = NP // d
    c2d, q2d = _tc_chain(NP, d)(
        dinv2x[:NP].reshape(R, d), sp.reshape(2, R, d))
    c_col = c2d.reshape(NP, 1)
    q_col = q2d.reshape(NP, 1)
    xws = _tc_scale_mm(NP, n, d)(x, W, c_col)
    srcc32h = srcc32.reshape(32, 2, NCH32 // 2, _CH)
    dstc32h = dstc32.reshape(32, 2, NCH32 // 2, _CH)
    aggp = _sc_aggregate(NP, NCH32, d)(srcc32h, dstc32h, xws)
    y = _tc_epilogue(NP, n, d)(
        aggp, xws, x, c_col, q_col,
        b.reshape(1, d), gamma.reshape(1, d), beta.reshape(1, d))
    return y


# speedup vs baseline: 1.4336x; 1.1437x over previous
import functools

import jax
import jax.numpy as jnp
from jax import lax
from jax.experimental import pallas as pl
from jax.experimental.pallas import tpu as pltpu
from jax.experimental.pallas import tpu_sc as plsc

_F32 = jnp.float32
_CH = 128
_NB = 4
_NB5 = 2


def _sc_mesh():
    return plsc.VectorSubcoreMesh(core_axis_name="c", subcore_axis_name="s")


def _zero_fill_1d(ref, nvec):
    def body(i, carry):
        ref[pl.ds(i * 16, 16)] = jnp.zeros((16,), _F32)
        return carry
    lax.fori_loop(0, nvec, body, 0)


def _newton_rsqrt(x):
    bits = lax.bitcast_convert_type(x, jnp.int32)
    seed = lax.bitcast_convert_type(
        0x5F3759DF - lax.shift_right_logical(bits, 1), _F32)
    y = seed * (1.5 - 0.5 * x * seed * seed)
    y = y * (1.5 - 0.5 * x * y * y)
    return y


def _sc_norm(NP, NCH16, NCH32):
    rpt = NP // 16

    @functools.partial(
        pl.kernel,
        mesh=_sc_mesh(),
        out_type=(jax.ShapeDtypeStruct((2 * NP,), _F32),
                  jax.ShapeDtypeStruct((2 * NP,), _F32)),
        scratch_types=[
            pltpu.VMEM((NCH16, _CH), jnp.int32),
            pltpu.VMEM((NCH32, _CH), jnp.int32),
            pltpu.VMEM((NCH32, _CH), jnp.int32),
            pltpu.VMEM((_CH,), _F32),
            pltpu.VMEM((rpt,), _F32),
            pltpu.VMEM((rpt,), _F32),
            [pltpu.VMEM((_CH,), _F32)] * _NB,
            pltpu.VMEM_SHARED((NP,), _F32),
            pltpu.VMEM_SHARED((NP,), _F32),
            [pltpu.SemaphoreType.DMA] * _NB,
        ],
    )
    def k(srcc16, srcc32, dstc32, dinv_out, sp_out,
          idxh, isrc, idst, ones_v, zeros_v, degv, vrows, acc, dinv_sp,
          sems):
        cid = lax.axis_index("c")
        sid = lax.axis_index("s")
        wid = sid * 2 + cid
        hid = cid * 16 + sid
        for t in range(_CH // 16):
            ones_v[pl.ds(t * 16, 16)] = jnp.ones((16,), _F32)
        _zero_fill_1d(zeros_v, rpt // 16)
        pltpu.sync_copy(zeros_v, acc.at[pl.ds(sid * rpt, rpt)])
        pltpu.sync_copy(srcc16.at[hid], idxh)
        pltpu.sync_copy(srcc32.at[wid], isrc)
        pltpu.sync_copy(dstc32.at[wid], idst)
        plsc.subcore_barrier()

        def hbody(j, carry):
            pltpu.sync_copy(ones_v, acc.at[idxh.at[j]], add=True)
            return carry
        lax.fori_loop(0, NCH16, hbody, 0)
        plsc.subcore_barrier()

        pltpu.sync_copy(acc.at[pl.ds(sid * rpt, rpt)], degv)

        def rbody(i, carry):
            x = degv[pl.ds(i * 16, 16)] + 1.0
            degv[pl.ds(i * 16, 16)] = _newton_rsqrt(x)
            return carry
        lax.fori_loop(0, rpt // 16, rbody, 0)
        pltpu.sync_copy(
            degv, dinv_out.at[pl.ds(cid * NP + sid * rpt, rpt)])
        pltpu.sync_copy(degv, dinv_sp.at[pl.ds(sid * rpt, rpt)])
        pltpu.sync_copy(zeros_v, acc.at[pl.ds(sid * rpt, rpt)])
        plsc.subcore_barrier()

        for b in range(_NB):
            pltpu.async_copy(dinv_sp.at[isrc.at[b]], vrows[b], sems[b])

        def sbody(q, carry):
            for b in range(_NB):
                e = q * _NB + b
                pltpu.make_async_copy(
                    dinv_sp.at[pl.ds(0, _CH)], vrows[b], sems[b]).wait()
                pltpu.sync_copy(vrows[b], acc.at[idst.at[e]], add=True)
                nxt = jnp.minimum(e + _NB, NCH32 - 1)
                pltpu.async_copy(
                    dinv_sp.at[isrc.at[nxt]], vrows[b], sems[b])
            return carry
        lax.fori_loop(0, NCH32 // _NB, sbody, 0)
        for b in range(_NB):
            pltpu.make_async_copy(
                dinv_out.at[pl.ds(0, _CH)], vrows[b], sems[b]).wait()
        plsc.subcore_barrier()
        pltpu.sync_copy(acc.at[pl.ds(sid * rpt, rpt)],
                        sp_out.at[pl.ds(cid * NP + sid * rpt, rpt)])

    return k


def _sc_aggregate(NP, NCH, D):
    rpt = NP // 16
    nzc = rpt // _CH
    HCH = NCH // 2

    @functools.partial(
        pl.kernel,
        mesh=_sc_mesh(),
        out_type=jax.ShapeDtypeStruct((2 * NP, D), _F32),
        compiler_params=pltpu.CompilerParams(use_tc_tiling_on_sc=False),
        scratch_types=[
            pltpu.VMEM((HCH, _CH), jnp.int32),
            pltpu.VMEM((HCH, _CH), jnp.int32),
            [pltpu.VMEM((_CH, D), _F32)] * 2,
            pltpu.VMEM_SHARED((NP, D), _F32),
            [pltpu.SemaphoreType.DMA] * 2,
        ],
    )
    def k(srcc, dstc, xws, out, isrc, idst, rows2, acc, gsems):
        cid = lax.axis_index("c")
        sid = lax.axis_index("s")
        wid = sid * 2 + cid

        def zrow(i, carry):
            for t in range(D // 16):
                rows2[0][i, pl.ds(t * 16, 16)] = jnp.zeros((16,), _F32)
            return carry
        lax.fori_loop(0, _CH, zrow, 0)
        for t in range(nzc):
            pltpu.sync_copy(
                rows2[0], acc.at[pl.ds(sid * rpt + t * _CH, _CH), :])
        plsc.subcore_barrier()

        for p in range(2):
            pltpu.sync_copy(srcc.at[wid, p], isrc)
            pltpu.sync_copy(dstc.at[wid, p], idst)
            for b in range(2):
                pltpu.async_copy(xws.at[isrc.at[b]], rows2[b], gsems[b])

            def body(q, carry):
                for b in range(2):
                    e = q * 2 + b
                    pltpu.make_async_copy(
                        xws.at[pl.ds(0, _CH), :], rows2[b],
                        gsems[b]).wait()
                    pltpu.sync_copy(rows2[b], acc.at[idst.at[e]], add=True)
                    nxt = jnp.minimum(e + 2, HCH - 1)
                    pltpu.async_copy(
                        xws.at[isrc.at[nxt]], rows2[b], gsems[b])
                return carry
            lax.fori_loop(0, HCH // 2, body, 0)
            for b in range(2):
                pltpu.make_async_copy(
                    xws.at[pl.ds(0, _CH), :], rows2[b], gsems[b]).wait()
        plsc.subcore_barrier()
        for t in range(nzc):
            pltpu.sync_copy(
                acc.at[pl.ds(sid * rpt + t * _CH, _CH), :],
                out.at[pl.ds(cid * NP + sid * rpt + t * _CH, _CH), :])

    return k


def _tc_chain(NP, D):
    R = NP // D

    def body(dinv_ref, sp_ref, c_ref, q_ref):
        dinv = dinv_ref[...]
        s = sp_ref[0] + sp_ref[1]
        deg2 = dinv * s + dinv * dinv + 2.0
        dinv2 = lax.rsqrt(deg2)
        c = dinv * dinv2
        c_ref[...] = c
        q_ref[...] = (c * c + 2.0 * dinv2 * dinv2) / c

    return pl.pallas_call(
        body,
        out_shape=(jax.ShapeDtypeStruct((R, D), _F32),
                   jax.ShapeDtypeStruct((R, D), _F32)),
    )


def _tc_scale_mm(NP, n, D):
    def body(x_ref, w_ref, c_ref, xws_ref):
        xw = lax.dot_general(
            x_ref[...], w_ref[...], (((1,), (1,)), ((), ())),
            precision=lax.Precision.HIGHEST,
            preferred_element_type=_F32)
        xws_ref[pl.ds(0, n), :] = xw * c_ref[pl.ds(0, n), :]
        xws_ref[pl.ds(n, NP - n), :] = jnp.zeros((NP - n, D), _F32)

    return pl.pallas_call(
        body,
        out_shape=jax.ShapeDtypeStruct((NP, D), _F32),
    )


def _tc_epilogue(NP, n, D):
    def body(agg_ref, xws_ref, x_ref, c_ref, q_ref,
             b_ref, g_ref, be_ref, y_ref):
        agg = agg_ref[pl.ds(0, n), :] + agg_ref[pl.ds(NP, n), :]
        out = (c_ref[pl.ds(0, n), :] * agg
               + q_ref[pl.ds(0, n), :] * xws_ref[pl.ds(0, n), :]
               + b_ref[...])
        mean = jnp.mean(out, axis=0, keepdims=True)
        var = jnp.mean((out - mean) ** 2, axis=0, keepdims=True)
        h = (out - mean) * lax.rsqrt(var + 1e-5) * g_ref[...] + be_ref[...]
        h = jnp.where(h > 0, h, 0.1 * h)
        y = h + x_ref[...]
        y_ref[...] = jnp.where(y > 0, y, 0.1 * y)

    return pl.pallas_call(
        body,
        out_shape=jax.ShapeDtypeStruct((n, D), _F32),
    )


def kernel(x, edge_index, W, b, gamma, beta):
    n, d = x.shape
    dh = d // 2
    e = edge_index.shape[1]
    grain32 = 32 * _CH * _NB
    EP = -(-e // grain32) * grain32
    NCH32 = EP // (32 * _CH)
    NCH16 = EP // (16 * _CH)
    NP = -(-(n + 16) // 256) * 256

    pad = EP - e
    padv = n + (jnp.arange(pad, dtype=jnp.int32) % 16)
    src = jnp.concatenate([edge_index[0], padv])
    dst = jnp.concatenate([edge_index[1], padv])
    srcc32 = src.reshape(32, NCH32, _CH)
    dstc32 = dst.reshape(32, NCH32, _CH)

    NCH16 = EP // (16 * _CH)
    src16 = src.reshape(1, 16, NCH16, _CH)
    srcc16 = jnp.concatenate([src16, src16], axis=0).reshape(
        32, NCH16, _CH)
    dinv2x, sp = _sc_norm(NP, NCH16, NCH32)(srcc16, srcc32, dstc32)
    R = NP // d
    c2d, q2d = _tc_chain(NP, d)(
        dinv2x[:NP].reshape(R, d), sp.reshape(2, R, d))
    c_col = c2d.reshape(NP, 1)
    q_col = q2d.reshape(NP, 1)
    xws = _tc_scale_mm(NP, n, d)(x, W, c_col)
    srcc32h = srcc32.reshape(32, 2, NCH32 // 2, _CH)
    dstc32h = dstc32.reshape(32, 2, NCH32 // 2, _CH)
    aggp = _sc_aggregate(NP, NCH32, d)(srcc32h, dstc32h, xws)
    y = _tc_epilogue(NP, n, d)(
        aggp, xws, x, c_col, q_col,
        b.reshape(1, d), gamma.reshape(1, d), beta.reshape(1, d))
    return y
